# baseline (speedup 1.0000x reference)
import jax
import jax.numpy as jnp
from jax.experimental import pallas as pl
from jax.experimental.pallas import tpu as pltpu

VOCAB = 1000
BV = 8


def _onehot_t_block(xt_ref, o_ref):
    i = pl.program_id(0)
    xt = xt_ref[...]
    v_idx = jax.lax.broadcasted_iota(
        jnp.int32, (xt_ref.shape[0], BV, xt_ref.shape[1]), 1
    ) + i * BV
    o_ref[...] = (xt[:, None, :] == v_idx).astype(jnp.float32)


def kernel(x):
    n0, n1 = x.shape
    xt = x.T
    out_t = pl.pallas_call(
        _onehot_t_block,
        grid=(VOCAB // BV,),
        in_specs=[pl.BlockSpec((n1, n0), lambda i: (0, 0))],
        out_specs=pl.BlockSpec((n1, BV, n0), lambda i: (0, i, 0)),
        out_shape=jax.ShapeDtypeStruct((n1, VOCAB, n0), jnp.float32),
        compiler_params=pltpu.CompilerParams(
            dimension_semantics=("parallel",),
            vmem_limit_bytes=100 * 1024 * 1024,
        ),
    )(xt)
    return out_t.transpose(2, 0, 1)


# speedup vs baseline: 1.2013x; 1.2013x over previous
import jax
import jax.numpy as jnp
from jax.experimental import pallas as pl
from jax.experimental.pallas import tpu as pltpu

VOCAB = 1000
BV = 40
NBUF = 3


def _onehot_t_ring(xt_ref, o_ref, vbuf, sems):
    n_steps = VOCAB // BV
    k, n = xt_ref.shape
    xt = xt_ref[...]
    iota = jax.lax.broadcasted_iota(jnp.int32, (k, BV, n), 1)

    def copy(i, slot):
        return pltpu.make_async_copy(
            vbuf.at[slot], o_ref.at[:, pl.ds(i * BV, BV), :], sems.at[slot]
        )

    def body(i, _):
        slot = jax.lax.rem(i, NBUF)

        @pl.when(i >= NBUF)
        def _():
            copy(i - NBUF, slot).wait()

        vbuf[slot] = (xt[:, None, :] == iota + i * BV).astype(jnp.float32)
        copy(i, slot).start()
        return ()

    jax.lax.fori_loop(0, n_steps, body, ())

    def drain(i, _):
        copy(i, jax.lax.rem(i, NBUF)).wait()
        return ()

    jax.lax.fori_loop(n_steps - NBUF, n_steps, drain, ())


def kernel(x):
    n0, n1 = x.shape
    xt = x.T
    out_t = pl.pallas_call(
        _onehot_t_ring,
        in_specs=[pl.BlockSpec(memory_space=pltpu.VMEM)],
        out_specs=pl.BlockSpec(memory_space=pl.ANY),
        out_shape=jax.ShapeDtypeStruct((n1, VOCAB, n0), jnp.float32),
        scratch_shapes=[
            pltpu.VMEM((NBUF, n1, BV, n0), jnp.float32),
            pltpu.SemaphoreType.DMA((NBUF,)),
        ],
        compiler_params=pltpu.CompilerParams(
            vmem_limit_bytes=100 * 1024 * 1024,
        ),
    )(xt)
    return out_t.transpose(2, 0, 1)


# speedup vs baseline: 1.2029x; 1.0013x over previous
import jax
import jax.numpy as jnp
from jax.experimental import pallas as pl
from jax.experimental.pallas import tpu as pltpu

VOCAB = 1000
BV = 40
BW = 8
NBUF = 3


def _onehot_t_ring(xt_ref, o_ref, vbuf, wsems, rsems):
    n_steps = VOCAB // BV
    n_warm = BV // BW
    k, n = xt_ref.shape
    xt = xt_ref[...]
    iota_w = jax.lax.broadcasted_iota(jnp.int32, (k, BW, n), 1)
    iota = jax.lax.broadcasted_iota(jnp.int32, (k, BV, n), 1)

    for j in range(n_warm):
        vbuf[0, :, j * BW:(j + 1) * BW, :] = (
            xt[:, None, :] == iota_w + j * BW
        ).astype(jnp.float32)
        pltpu.make_async_copy(
            vbuf.at[0, :, pl.ds(j * BW, BW), :],
            o_ref.at[:, pl.ds(j * BW, BW), :],
            wsems.at[j],
        ).start()

    def rcopy(i, slot):
        return pltpu.make_async_copy(
            vbuf.at[slot], o_ref.at[:, pl.ds(i * BV, BV), :], rsems.at[slot]
        )

    def body(i, _):
        slot = jax.lax.rem(i, NBUF)

        @pl.when(i >= NBUF + 1)
        def _():
            rcopy(i - NBUF, slot).wait()

        @pl.when(i == NBUF)
        def _():
            for j in range(n_warm):
                pltpu.make_async_copy(
                    vbuf.at[0, :, pl.ds(j * BW, BW), :],
                    o_ref.at[:, pl.ds(j * BW, BW), :],
                    wsems.at[j],
                ).wait()

        vbuf[slot] = (xt[:, None, :] == iota + i * BV).astype(jnp.float32)
        rcopy(i, slot).start()
        return ()

    jax.lax.fori_loop(1, n_steps, body, ())

    def drain(i, _):
        rcopy(i, jax.lax.rem(i, NBUF)).wait()
        return ()

    jax.lax.fori_loop(n_steps - NBUF, n_steps, drain, ())


def kernel(x):
    n0, n1 = x.shape
    xt = x.T
    out_t = pl.pallas_call(
        _onehot_t_ring,
        in_specs=[pl.BlockSpec(memory_space=pltpu.VMEM)],
        out_specs=pl.BlockSpec(memory_space=pl.ANY),
        out_shape=jax.ShapeDtypeStruct((n1, VOCAB, n0), jnp.float32),
        scratch_shapes=[
            pltpu.VMEM((NBUF, n1, BV, n0), jnp.float32),
            pltpu.SemaphoreType.DMA((BV // BW,)),
            pltpu.SemaphoreType.DMA((NBUF,)),
        ],
        compiler_params=pltpu.CompilerParams(
            vmem_limit_bytes=100 * 1024 * 1024,
        ),
    )(xt)
    return out_t.transpose(2, 0, 1)
